# Initial kernel scaffold; baseline (speedup 1.0000x reference)
import jax
import jax.numpy as jnp
from jax.experimental import pallas as pl


def kernel(positions, cell, numbers):
    raise NotImplementedError("write your pallas kernel here")


# speedup vs baseline: 189.3150x; 189.3150x over previous
import jax
import jax.numpy as jnp
import numpy as np
from jax.experimental import pallas as pl
from jax.experimental.pallas import tpu as pltpu

_N = 256
_K = 17
_REPL = 3
_NCELLS = _REPL ** 3
_NTOT = _NCELLS * _N
_NEXT = 128
_NFULL = _NTOT + _NEXT
_CENTER_CELL = (_NCELLS - 1) // 2
_START = _CENTER_CELL * _N


def _round_bf16(x):
    return x.astype(jnp.bfloat16).astype(jnp.float32)


def _dist_kernel(qpos_ref, kpost_ref, keepadd_ref, qidx_ref, padadd_ref,
                 full_ref):
    qpos = qpos_ref[...]
    kpost = kpost_ref[...]
    qx, qy, qz = qpos[:, 0:1], qpos[:, 1:2], qpos[:, 2:3]
    kx, ky, kz = kpost[0:1, :], kpost[1:2, :], kpost[2:3, :]
    sqq = (qx * qx + qz * qz) + qy * qy
    sqk = (kx * kx + kz * kz) + ky * ky
    dot = (_round_bf16(qx) * _round_bf16(kx)
           + _round_bf16(qy) * _round_bf16(ky)) \
        + _round_bf16(qz) * _round_bf16(kz)
    d = (sqq + sqk) - 2.0 * dot
    d0 = d[:, 0:1]
    cols = jax.lax.broadcasted_iota(jnp.int32, (_N, _NTOT), 1)
    dm = d + keepadd_ref[...]
    dm = jnp.where(cols == qidx_ref[...], jnp.inf, dm)
    ext = d0 + padadd_ref[...]
    full_ref[...] = jnp.concatenate([dm, ext], axis=1)


def _topk_kernel(full_ref, src_ref, dist_ref):
    full = full_ref[...]
    iota = jax.lax.broadcasted_iota(jnp.int32, (_N, _NFULL), 1)
    ms, idxs = [], []
    for _ in range(_K):
        m = jnp.min(full, axis=1, keepdims=True)
        hit = full == m
        idx = jnp.min(jnp.where(hit, iota, _NFULL), axis=1, keepdims=True)
        ms.append(m)
        idxs.append(idx)
        full = jnp.where(iota == idx, jnp.inf, full)
    mv = jnp.concatenate(ms, axis=1)
    iv = jnp.concatenate(idxs, axis=1)
    src_ref[...] = jnp.where(iv < _NTOT, jnp.bitwise_and(iv, _N - 1), 0)
    dist_ref[...] = jnp.sqrt(jnp.maximum(mv, 0.0) + 1e-12)


def kernel(positions, cell, numbers):
    ii = jnp.arange(_REPL, dtype=positions.dtype)
    grid = jnp.stack(jnp.meshgrid(ii, ii, ii, indexing="ij"), axis=-1).reshape(-1, 3)
    shifts = grid @ cell
    supercell = (positions[None, :, :] + shifts[:, None, :]).reshape(-1, 3)
    frac = supercell @ jnp.linalg.inv(cell)
    center = (_REPL - 1) / 2.0
    filt = jnp.all((frac >= center - 1.0 - 1e-5) & (frac < center + 2.0 + 1e-5), axis=1)
    idx_keep = jnp.nonzero(filt, size=_NTOT, fill_value=0)[0]
    nkept = jnp.sum(filt.astype(jnp.int32))
    q_idx = idx_keep[_START:_START + _N].astype(jnp.int32)
    qpos = supercell[q_idx]
    kpost = supercell.T
    keepadd = jnp.where(filt, 0.0, jnp.inf).astype(jnp.float32).reshape(1, _NTOT)
    padadd = jnp.where(jnp.arange(_NEXT, dtype=jnp.int32) < (_NTOT - nkept),
                       0.0, jnp.inf).astype(jnp.float32).reshape(1, _NEXT)

    full = pl.pallas_call(
        _dist_kernel,
        out_shape=jax.ShapeDtypeStruct((_N, _NFULL), jnp.float32),
    )(qpos, kpost, keepadd, q_idx.reshape(_N, 1), padadd)
    src_mat, dist_mat = pl.pallas_call(
        _topk_kernel,
        out_shape=(jax.ShapeDtypeStruct((_N, _K), jnp.int32),
                   jax.ShapeDtypeStruct((_N, _K), jnp.float32)),
    )(full)

    src_atom = src_mat.reshape(-1)
    edge_dist = dist_mat.reshape(-1)
    dst_atom = jnp.repeat(jnp.arange(_N, dtype=jnp.int32), _K)
    return (src_atom, dst_atom, edge_dist, numbers, jnp.asarray(0))


# speedup vs baseline: 197.1308x; 1.0413x over previous
import jax
import jax.numpy as jnp
from jax.experimental import pallas as pl
from jax.experimental.pallas import tpu as pltpu

_N = 256
_K = 17
_REPL = 3
_NCELLS = _REPL ** 3
_NTOT = _NCELLS * _N
_NEXT = 128
_NFULL = _NTOT + _NEXT
_CENTER_CELL = (_NCELLS - 1) // 2
_START = _CENTER_CELL * _N


def _round_bf16(x):
    return x.astype(jnp.bfloat16).astype(jnp.float32)


def _fused_kernel(qpos_ref, kpost_ref, keepadd_ref, qidx_ref, padadd_ref,
                  src_ref, dist_ref, full_ref):
    qpos = qpos_ref[...]
    kpost = kpost_ref[...]
    qx, qy, qz = qpos[:, 0:1], qpos[:, 1:2], qpos[:, 2:3]
    kx, ky, kz = kpost[0:1, :], kpost[1:2, :], kpost[2:3, :]
    sqq = (qx * qx + qz * qz) + qy * qy
    sqk = (kx * kx + kz * kz) + ky * ky
    dot = (_round_bf16(qx) * _round_bf16(kx)
           + _round_bf16(qy) * _round_bf16(ky)) \
        + _round_bf16(qz) * _round_bf16(kz)
    d = (sqq + sqk) - 2.0 * dot
    d0 = d[:, 0:1]
    cols = jax.lax.broadcasted_iota(jnp.int32, (_N, _NTOT), 1)
    dm = d + keepadd_ref[...]
    dm = jnp.where(cols == qidx_ref[...], jnp.inf, dm)
    ext = d0 + padadd_ref[...]
    full_ref[...] = jnp.concatenate([dm, ext], axis=1)
    full = full_ref[...]
    iota = jax.lax.broadcasted_iota(jnp.int32, (_N, _NFULL), 1)
    ms, idxs = [], []
    for _ in range(_K):
        m = jnp.min(full, axis=1, keepdims=True)
        hit = full == m
        idx = jnp.min(jnp.where(hit, iota, _NFULL), axis=1, keepdims=True)
        ms.append(m)
        idxs.append(idx)
        full = jnp.where(iota == idx, jnp.inf, full)
    mv = jnp.concatenate(ms, axis=1)
    iv = jnp.concatenate(idxs, axis=1)
    src_ref[...] = jnp.where(iv < _NTOT, jnp.bitwise_and(iv, _N - 1), 0)
    dist_ref[...] = jnp.sqrt(jnp.maximum(mv, 0.0) + 1e-12)


def kernel(positions, cell, numbers):
    ii = jnp.arange(_REPL, dtype=positions.dtype)
    grid = jnp.stack(jnp.meshgrid(ii, ii, ii, indexing="ij"), axis=-1).reshape(-1, 3)
    shifts = grid @ cell
    supercell = (positions[None, :, :] + shifts[:, None, :]).reshape(-1, 3)
    frac = supercell @ jnp.linalg.inv(cell)
    center = (_REPL - 1) / 2.0
    filt = jnp.all((frac >= center - 1.0 - 1e-5) & (frac < center + 2.0 + 1e-5), axis=1)
    idx_keep = jnp.nonzero(filt, size=_NTOT, fill_value=0)[0]
    nkept = jnp.sum(filt.astype(jnp.int32))
    q_idx = idx_keep[_START:_START + _N].astype(jnp.int32)
    qpos = supercell[q_idx]
    kpost = supercell.T
    keepadd = jnp.where(filt, 0.0, jnp.inf).astype(jnp.float32).reshape(1, _NTOT)
    padadd = jnp.where(jnp.arange(_NEXT, dtype=jnp.int32) < (_NTOT - nkept),
                       0.0, jnp.inf).astype(jnp.float32).reshape(1, _NEXT)
    src_mat, dist_mat = pl.pallas_call(
        _fused_kernel,
        out_shape=(jax.ShapeDtypeStruct((_N, _K), jnp.int32),
                   jax.ShapeDtypeStruct((_N, _K), jnp.float32)),
        scratch_shapes=[pltpu.VMEM((_N, _NFULL), jnp.float32)],
    )(qpos, kpost, keepadd, q_idx.reshape(_N, 1), padadd)
    src_atom = src_mat.reshape(-1)
    edge_dist = dist_mat.reshape(-1)
    dst_atom = jnp.repeat(jnp.arange(_N, dtype=jnp.int32), _K)
    return (src_atom, dst_atom, edge_dist, numbers, jnp.asarray(0))


# speedup vs baseline: 197.1923x; 1.0003x over previous
import jax
import jax.numpy as jnp
from jax.experimental import pallas as pl
from jax.experimental.pallas import tpu as pltpu

_N = 256
_K = 17
_REPL = 3
_NCELLS = _REPL ** 3
_NTOT = _NCELLS * _N
_NEXT = 128
_NFULL = _NTOT + _NEXT
_CENTER_CELL = (_NCELLS - 1) // 2
_START = _CENTER_CELL * _N


def _round_bf16(x):
    return x.astype(jnp.bfloat16).astype(jnp.float32)


def _fused_kernel(qpos_ref, kpost_ref, keepadd_ref, qidx_ref, padadd_ref,
                  src_ref, dist_ref, full_ref):
    qpos = qpos_ref[...]
    kpost = kpost_ref[...]
    qx, qy, qz = qpos[:, 0:1], qpos[:, 1:2], qpos[:, 2:3]
    kx, ky, kz = kpost[0:1, :], kpost[1:2, :], kpost[2:3, :]
    sqq = (qx * qx + qz * qz) + qy * qy
    sqk = (kx * kx + kz * kz) + ky * ky
    dot = (_round_bf16(qx) * _round_bf16(kx)
           + _round_bf16(qy) * _round_bf16(ky)) \
        + _round_bf16(qz) * _round_bf16(kz)
    d = (sqq + sqk) - 2.0 * dot
    d0 = d[:, 0:1]
    cols = jax.lax.broadcasted_iota(jnp.int32, (_N, _NTOT), 1)
    dm = d + keepadd_ref[...]
    dm = jnp.where(cols == qidx_ref[...], jnp.inf, dm)
    ext = d0 + padadd_ref[...]
    full_ref[...] = jnp.concatenate([dm, ext], axis=1)
    full = full_ref[...]
    iota = jax.lax.broadcasted_iota(jnp.int32, (_N, _NFULL), 1)
    ms, idxs = [], []
    for t in range(_K):
        m = jnp.min(full, axis=1, keepdims=True)
        hit = full == m
        idx = jnp.min(jnp.where(hit, iota, _NFULL), axis=1, keepdims=True)
        ms.append(m)
        idxs.append(idx)
        if t + 1 < _K:
            full = jnp.where(iota == idx, jnp.inf, full)
    mv = jnp.concatenate(ms, axis=1)
    iv = jnp.concatenate(idxs, axis=1)
    src_ref[...] = jnp.where(iv < _NTOT, jnp.bitwise_and(iv, _N - 1), 0)
    dist_ref[...] = jnp.sqrt(jnp.maximum(mv, 0.0) + 1e-12)


def kernel(positions, cell, numbers):
    ii = jnp.arange(_REPL, dtype=positions.dtype)
    grid = jnp.stack(jnp.meshgrid(ii, ii, ii, indexing="ij"), axis=-1).reshape(-1, 3)
    shifts = grid @ cell
    supercell = (positions[None, :, :] + shifts[:, None, :]).reshape(-1, 3)
    frac = supercell @ jnp.linalg.inv(cell)
    center = (_REPL - 1) / 2.0
    filt = jnp.all((frac >= center - 1.0 - 1e-5) & (frac < center + 2.0 + 1e-5), axis=1)
    idx_keep = jnp.nonzero(filt, size=_NTOT, fill_value=0)[0]
    nkept = jnp.sum(filt.astype(jnp.int32))
    q_idx = idx_keep[_START:_START + _N].astype(jnp.int32)
    qpos = supercell[q_idx]
    kpost = supercell.T
    keepadd = jnp.where(filt, 0.0, jnp.inf).astype(jnp.float32).reshape(1, _NTOT)
    padadd = jnp.where(jnp.arange(_NEXT, dtype=jnp.int32) < (_NTOT - nkept),
                       0.0, jnp.inf).astype(jnp.float32).reshape(1, _NEXT)
    src_mat, dist_mat = pl.pallas_call(
        _fused_kernel,
        out_shape=(jax.ShapeDtypeStruct((_N, _K), jnp.int32),
                   jax.ShapeDtypeStruct((_N, _K), jnp.float32)),
        scratch_shapes=[pltpu.VMEM((_N, _NFULL), jnp.float32)],
    )(qpos, kpost, keepadd, q_idx.reshape(_N, 1), padadd)
    src_atom = src_mat.reshape(-1)
    edge_dist = dist_mat.reshape(-1)
    dst_atom = jnp.repeat(jnp.arange(_N, dtype=jnp.int32), _K)
    return (src_atom, dst_atom, edge_dist, numbers, jnp.asarray(0))
